# baseline (speedup 1.0000x reference)
import functools

import jax
import jax.numpy as jnp
from jax import lax
from jax.experimental import pallas as pl
from jax.experimental.pallas import tpu as pltpu
from jax.experimental.pallas import tpu_sc as plsc

_NUM_WORKERS = 32
_NBUF = 8


def _pool_sc(inp, lengths, table):
    maxlen, batch = inp.shape
    _, dim = table.shape
    bpw = batch // _NUM_WORKERS

    mesh = plsc.VectorSubcoreMesh(core_axis_name="c", subcore_axis_name="s")

    @functools.partial(
        pl.kernel,
        mesh=mesh,
        out_type=(
            jax.ShapeDtypeStruct((batch, dim), jnp.float32),
            jax.ShapeDtypeStruct((batch, dim), jnp.float32),
        ),
        compiler_params=pltpu.CompilerParams(use_tc_tiling_on_sc=False),
        scratch_types=(
            [
                pltpu.VMEM((maxlen, bpw), jnp.int32),
                pltpu.VMEM((bpw,), jnp.int32),
                pltpu.VMEM((bpw, dim), jnp.float32),
            ]
            + [pltpu.VMEM((bpw, dim), jnp.float32) for _ in range(_NBUF)]
            + [pltpu.SemaphoreType.DMA for _ in range(_NBUF)]
        ),
    )
    def k(inp_hbm, len_hbm, table_hbm, out_hbm, dummy_hbm,
          idx_v, lens_v, acc_v, *rest):
        rows = rest[:_NBUF]
        sems = rest[_NBUF:]
        wid = lax.axis_index("c") * 16 + lax.axis_index("s")
        base = wid * bpw

        pltpu.sync_copy(inp_hbm.at[:, pl.ds(base, bpw)], idx_v)
        pltpu.sync_copy(len_hbm.at[pl.ds(base, bpw)], lens_v)

        zeros_f = jnp.zeros((16,), jnp.float32)

        @pl.loop(0, bpw // 16)
        def _(j):
            lv = lens_v[pl.ds(j * 16, 16)]
            dv = idx_v[0, pl.ds(j * 16, 16)]

            @pl.loop(1, maxlen, unroll=4)
            def _(t):
                iv = idx_v[t, pl.ds(j * 16, 16)]
                idx_v[t, pl.ds(j * 16, 16)] = jnp.where(lv > t, iv, dv)

        @pl.loop(0, bpw, unroll=4)
        def _(i):
            @pl.loop(0, dim // 16)
            def _(j):
                acc_v[i, pl.ds(j * 16, 16)] = zeros_f

        def start(t, buf, sem):
            pltpu.make_async_copy(table_hbm.at[idx_v.at[t]], buf, sem).start()

        def finish(t, buf, sem):
            pltpu.make_async_copy(table_hbm.at[idx_v.at[t]], buf, sem).wait()

        def accum(buf):
            @pl.loop(0, bpw, unroll=4)
            def _(i):
                for j in range(dim // 16):
                    sl = pl.ds(j * 16, 16)
                    plsc.addupdate(acc_v.at[i, sl], buf[i, sl])

        for b in range(_NBUF):
            start(b, rows[b], sems[b])

        @pl.loop(0, maxlen, step=_NBUF)
        def _(t):
            for b in range(_NBUF):
                finish(t + b, rows[b], sems[b])
                accum(rows[b])

                @pl.when(t + b == 0)
                def _(b=b):
                    pltpu.sync_copy(rows[b], dummy_hbm.at[pl.ds(base, bpw)])

                @pl.when(t + b + _NBUF < maxlen)
                def _(b=b):
                    start(t + b + _NBUF, rows[b], sems[b])

        pltpu.sync_copy(acc_v, out_hbm.at[pl.ds(base, bpw)])

    return k(inp, lengths, table)


_TBLK = 2048


def _transpose_body(tt_ref, out_ref):
    x = tt_ref[...]
    y = x.reshape(32, _TBLK // 4, 4)
    out_ref[...] = y.transpose(1, 2, 0).reshape(_TBLK // 4, 128)


def _table_rowmajor(table):
    vocab, dim = table.shape
    nblk = (vocab + _TBLK - 1) // _TBLK
    out = pl.pallas_call(
        _transpose_body,
        grid=(nblk,),
        in_specs=[pl.BlockSpec((dim, _TBLK), lambda j: (0, j))],
        out_specs=pl.BlockSpec((_TBLK // 4, 128), lambda j: (j, 0)),
        out_shape=jax.ShapeDtypeStruct((vocab * dim // 128, 128), jnp.float32),
    )(table.T)
    return out.reshape(vocab, dim)


def _ffnn_body(maxlen, sums_ref, len_ref, dummy_ref, w1_ref, b1_ref,
               w2_ref, b2_ref, out_ref):
    lf = len_ref[...].astype(jnp.float32)
    sums = sums_ref[...] - (maxlen - lf) * dummy_ref[...]
    vec = sums / jnp.maximum(lf, 1.0)
    h = jnp.dot(vec, w1_ref[...], preferred_element_type=jnp.float32)
    h = jnp.maximum(h + b1_ref[...], 0.0)
    logits = jnp.dot(h, w2_ref[...], preferred_element_type=jnp.float32)
    logits = logits + b2_ref[...]
    m = jnp.max(logits, axis=1, keepdims=True)
    lse = jnp.log(jnp.sum(jnp.exp(logits - m), axis=1, keepdims=True)) + m
    out_ref[...] = logits - lse


def kernel(inp, lengths, table, W1, b1, W2, b2):
    maxlen, batch = inp.shape
    out_dim = W2.shape[1]

    sums, dummy = _pool_sc(inp.astype(jnp.int32), lengths.astype(jnp.int32),
                           _table_rowmajor(table))

    return pl.pallas_call(
        functools.partial(_ffnn_body, float(maxlen)),
        out_shape=jax.ShapeDtypeStruct((batch, out_dim), jnp.float32),
    )(sums, lengths.reshape(batch, 1), dummy,
      W1, b1.reshape(1, -1), W2, b2.reshape(1, -1))


# speedup vs baseline: 2.2787x; 2.2787x over previous
import functools

import jax
import jax.numpy as jnp
from jax import lax
from jax.experimental import pallas as pl
from jax.experimental.pallas import tpu as pltpu
from jax.experimental.pallas import tpu_sc as plsc

_NUM_WORKERS = 32
_NBUF = 8


def _pool_sc(inp, lengths, table):
    maxlen, batch = inp.shape
    _, dim = table.shape
    bpw = batch // _NUM_WORKERS

    mesh = plsc.VectorSubcoreMesh(core_axis_name="c", subcore_axis_name="s")

    @functools.partial(
        pl.kernel,
        mesh=mesh,
        out_type=(
            jax.ShapeDtypeStruct((batch, dim), jnp.float32),
            jax.ShapeDtypeStruct((batch, dim), jnp.float32),
        ),
        compiler_params=pltpu.CompilerParams(use_tc_tiling_on_sc=False),
        scratch_types=(
            [
                pltpu.VMEM((maxlen, bpw), jnp.int32),
                pltpu.VMEM((bpw,), jnp.int32),
                pltpu.VMEM((bpw, dim), jnp.float32),
            ]
            + [pltpu.VMEM((bpw, dim), jnp.float32) for _ in range(_NBUF)]
            + [pltpu.SemaphoreType.DMA for _ in range(_NBUF)]
        ),
    )
    def k(inp_hbm, len_hbm, table_hbm, out_hbm, dummy_hbm,
          idx_v, lens_v, acc_v, *rest):
        rows = rest[:_NBUF]
        sems = rest[_NBUF:]
        wid = lax.axis_index("c") * 16 + lax.axis_index("s")
        base = wid * bpw

        pltpu.sync_copy(inp_hbm.at[:, pl.ds(base, bpw)], idx_v)
        pltpu.sync_copy(len_hbm.at[pl.ds(base, bpw)], lens_v)

        zeros_f = jnp.zeros((16,), jnp.float32)

        @pl.loop(0, bpw // 16)
        def _(j):
            lv = lens_v[pl.ds(j * 16, 16)]
            dv = idx_v[0, pl.ds(j * 16, 16)]

            @pl.loop(1, maxlen, unroll=4)
            def _(t):
                iv = idx_v[t, pl.ds(j * 16, 16)]
                idx_v[t, pl.ds(j * 16, 16)] = jnp.where(lv > t, iv, dv)

        @pl.loop(0, bpw, unroll=4)
        def _(i):
            @pl.loop(0, dim // 16)
            def _(j):
                acc_v[i, pl.ds(j * 16, 16)] = zeros_f

        def start(t, buf, sem):
            pltpu.make_async_copy(table_hbm.at[idx_v.at[t]], buf, sem).start()

        def finish(t, buf, sem):
            pltpu.make_async_copy(table_hbm.at[idx_v.at[t]], buf, sem).wait()

        def accum(buf):
            @pl.loop(0, bpw, unroll=4)
            def _(i):
                for j in range(dim // 16):
                    sl = pl.ds(j * 16, 16)
                    plsc.addupdate(acc_v.at[i, sl], buf[i, sl])

        for b in range(_NBUF):
            start(b, rows[b], sems[b])

        @pl.loop(0, maxlen, step=_NBUF)
        def _(t):
            for b in range(_NBUF):
                finish(t + b, rows[b], sems[b])
                accum(rows[b])

                @pl.when(t + b == 0)
                def _(b=b):
                    pltpu.sync_copy(rows[b], dummy_hbm.at[pl.ds(base, bpw)])

                @pl.when(t + b + _NBUF < maxlen)
                def _(b=b):
                    start(t + b + _NBUF, rows[b], sems[b])

        pltpu.sync_copy(acc_v, out_hbm.at[pl.ds(base, bpw)])

    return k(inp, lengths, table)


def _table_rowmajor(table):
    vocab, dim = table.shape
    n_full = vocab // 128
    has_tail = vocab % 128 != 0
    tail_owner = (n_full - 1) % _NUM_WORKERS
    mesh = plsc.VectorSubcoreMesh(core_axis_name="c", subcore_axis_name="s")

    @functools.partial(
        pl.kernel,
        mesh=mesh,
        out_type=jax.ShapeDtypeStruct((vocab * dim // 128, 128), jnp.float32),
        compiler_params=pltpu.CompilerParams(use_tc_tiling_on_sc=True,
                                             needs_layout_passes=False),
        scratch_types=[
            pltpu.VMEM((dim, 128), jnp.float32),
            pltpu.VMEM((dim, 128), jnp.float32),
            pltpu.SemaphoreType.DMA,
        ],
    )
    def k(tt_hbm, tail_hbm, out_hbm, stage_v, out_v, sem):
        wid = lax.axis_index("c") * 16 + lax.axis_index("s")
        iota = lax.iota(jnp.int32, 16)

        def transpose_tile():
            @pl.loop(0, 128, unroll=4)
            def _(r):
                rs = jnp.full((16,), r, jnp.int32)
                row = r >> 2
                col = (r & 3) * dim
                for h in range(dim // 16):
                    v = plsc.load_gather(stage_v, [iota + h * 16, rs])
                    out_v[row, pl.ds(col + h * 16, 16)] = v

        @pl.loop(wid, n_full, step=_NUM_WORKERS)
        def _(j):
            pltpu.async_copy(tt_hbm.at[:, pl.ds(j * 128, 128)], stage_v,
                             sem).wait()
            transpose_tile()
            pltpu.async_copy(out_v, out_hbm.at[pl.ds(j * dim, dim)],
                             sem).wait()

        if has_tail:
            @pl.when(wid == tail_owner)
            def _():
                pltpu.async_copy(tail_hbm, stage_v, sem).wait()
                transpose_tile()
                pltpu.async_copy(
                    out_v, out_hbm.at[pl.ds((vocab - 128) * dim // 128, dim)],
                    sem).wait()

    tail_block = lax.slice(table.T, (0, vocab - 128), (dim, vocab))
    return k(table.T, tail_block).reshape(vocab, dim)


def _ffnn_body(maxlen, sums_ref, len_ref, dummy_ref, w1_ref, b1_ref,
               w2_ref, b2_ref, out_ref):
    lf = len_ref[...].astype(jnp.float32)
    sums = sums_ref[...] - (maxlen - lf) * dummy_ref[...]
    vec = sums / jnp.maximum(lf, 1.0)
    h = jnp.dot(vec, w1_ref[...], preferred_element_type=jnp.float32)
    h = jnp.maximum(h + b1_ref[...], 0.0)
    logits = jnp.dot(h, w2_ref[...], preferred_element_type=jnp.float32)
    logits = logits + b2_ref[...]
    m = jnp.max(logits, axis=1, keepdims=True)
    lse = jnp.log(jnp.sum(jnp.exp(logits - m), axis=1, keepdims=True)) + m
    out_ref[...] = logits - lse


def kernel(inp, lengths, table, W1, b1, W2, b2):
    maxlen, batch = inp.shape
    out_dim = W2.shape[1]

    sums, dummy = _pool_sc(inp.astype(jnp.int32), lengths.astype(jnp.int32),
                           _table_rowmajor(table))

    return pl.pallas_call(
        functools.partial(_ffnn_body, float(maxlen)),
        out_shape=jax.ShapeDtypeStruct((batch, out_dim), jnp.float32),
    )(sums, lengths.reshape(batch, 1), dummy,
      W1, b1.reshape(1, -1), W2, b2.reshape(1, -1))


# speedup vs baseline: 2.8319x; 1.2428x over previous
import functools

import jax
import jax.numpy as jnp
from jax import lax
from jax.experimental import pallas as pl
from jax.experimental.pallas import tpu as pltpu
from jax.experimental.pallas import tpu_sc as plsc

_NUM_WORKERS = 32
_NBUF = 8


def _pool_sc(inp, lengths, table):
    maxlen, batch = inp.shape
    _, dim = table.shape
    bpw = batch // _NUM_WORKERS

    mesh = plsc.VectorSubcoreMesh(core_axis_name="c", subcore_axis_name="s")

    @functools.partial(
        pl.kernel,
        mesh=mesh,
        out_type=(
            jax.ShapeDtypeStruct((batch, dim), jnp.float32),
            jax.ShapeDtypeStruct((batch, dim), jnp.float32),
        ),
        compiler_params=pltpu.CompilerParams(use_tc_tiling_on_sc=False),
        scratch_types=(
            [
                pltpu.VMEM((maxlen, bpw), jnp.int32),
                pltpu.VMEM((bpw,), jnp.int32),
                pltpu.VMEM((bpw, dim), jnp.float32),
            ]
            + [pltpu.VMEM((bpw, dim), jnp.float32) for _ in range(_NBUF)]
            + [pltpu.SemaphoreType.DMA for _ in range(_NBUF)]
        ),
    )
    def k(inp_hbm, len_hbm, table_hbm, out_hbm, dummy_hbm,
          idx_v, lens_v, acc_v, *rest):
        rows = rest[:_NBUF]
        sems = rest[_NBUF:]
        wid = lax.axis_index("c") * 16 + lax.axis_index("s")
        base = wid * bpw

        pltpu.sync_copy(inp_hbm.at[:, pl.ds(base, bpw)], idx_v)
        pltpu.sync_copy(len_hbm.at[pl.ds(base, bpw)], lens_v)

        zeros_f = jnp.zeros((16,), jnp.float32)

        @pl.loop(0, bpw // 16)
        def _(j):
            lv = lens_v[pl.ds(j * 16, 16)]
            dv = idx_v[0, pl.ds(j * 16, 16)]

            @pl.loop(1, maxlen, unroll=4)
            def _(t):
                iv = idx_v[t, pl.ds(j * 16, 16)]
                idx_v[t, pl.ds(j * 16, 16)] = jnp.where(lv > t, iv, dv)

        @pl.loop(0, bpw, unroll=4)
        def _(i):
            @pl.loop(0, dim // 16)
            def _(j):
                acc_v[i, pl.ds(j * 16, 16)] = zeros_f

        def start(t, buf, sem):
            pltpu.make_async_copy(table_hbm.at[idx_v.at[t]], buf, sem).start()

        def finish(t, buf, sem):
            pltpu.make_async_copy(table_hbm.at[idx_v.at[t]], buf, sem).wait()

        def accum(buf):
            @pl.loop(0, bpw, unroll=4)
            def _(i):
                for j in range(dim // 16):
                    sl = pl.ds(j * 16, 16)
                    plsc.addupdate(acc_v.at[i, sl], buf[i, sl])

        for b in range(_NBUF):
            start(b, rows[b], sems[b])

        @pl.loop(0, maxlen, step=_NBUF)
        def _(t):
            for b in range(_NBUF):
                finish(t + b, rows[b], sems[b])
                accum(rows[b])

                @pl.when(t + b == 0)
                def _(b=b):
                    pltpu.sync_copy(rows[b], dummy_hbm.at[pl.ds(base, bpw)])

                @pl.when(t + b + _NBUF < maxlen)
                def _(b=b):
                    start(t + b + _NBUF, rows[b], sems[b])

        pltpu.sync_copy(acc_v, out_hbm.at[pl.ds(base, bpw)])

    return k(inp, lengths, table)


def _table_rowmajor(table):
    vocab, dim = table.shape
    blk = 512
    n_blk = vocab // blk
    has_tail = vocab % blk != 0
    tail_owner = (n_blk - 1) % _NUM_WORKERS
    o_rows = blk * dim // 128
    mesh = plsc.VectorSubcoreMesh(core_axis_name="c", subcore_axis_name="s")

    @functools.partial(
        pl.kernel,
        mesh=mesh,
        out_type=jax.ShapeDtypeStruct((vocab * dim // 128, 128), jnp.float32),
        compiler_params=pltpu.CompilerParams(use_tc_tiling_on_sc=True,
                                             needs_layout_passes=False),
        scratch_types=[
            pltpu.VMEM((dim, blk), jnp.float32),
            pltpu.VMEM((dim, blk), jnp.float32),
            pltpu.VMEM((o_rows, 128), jnp.float32),
            pltpu.VMEM((o_rows, 128), jnp.float32),
            pltpu.SemaphoreType.DMA,
            pltpu.SemaphoreType.DMA,
            pltpu.SemaphoreType.DMA,
            pltpu.SemaphoreType.DMA,
        ],
    )
    def k(tt_hbm, tail_hbm, out_hbm, s0, s1, o0, o1, si0, si1, so0, so1):
        wid = lax.axis_index("c") * 16 + lax.axis_index("s")
        iota = lax.iota(jnp.int32, 16)

        def issue_in(j, sv, sem):
            pltpu.make_async_copy(tt_hbm.at[:, pl.ds(j * blk, blk)], sv,
                                  sem).start()

        def wait_in(sv, sem):
            pltpu.make_async_copy(tt_hbm.at[:, pl.ds(0, blk)], sv, sem).wait()

        def transpose_blk(sv, ov):
            @pl.loop(0, blk, unroll=8)
            def _(r):
                rs = jnp.full((16,), r, jnp.int32)
                row = r >> 2
                col = (r & 3) * dim
                for h in range(dim // 16):
                    v = plsc.load_gather(sv, [iota + h * 16, rs])
                    ov[row, pl.ds(col + h * 16, 16)] = v

        def issue_out(j, ov, sem):
            pltpu.make_async_copy(ov, out_hbm.at[pl.ds(j * o_rows, o_rows)],
                                  sem).start()

        def wait_out(ov, sem):
            pltpu.make_async_copy(ov, out_hbm.at[pl.ds(0, o_rows)],
                                  sem).wait()

        issue_in(wid, s0, si0)

        @pl.loop(wid, n_blk, step=2 * _NUM_WORKERS)
        def _(j):
            @pl.when(j + _NUM_WORKERS < n_blk)
            def _():
                issue_in(j + _NUM_WORKERS, s1, si1)

            wait_in(s0, si0)
            transpose_blk(s0, o0)
            issue_out(j, o0, so0)

            @pl.when(j + 2 * _NUM_WORKERS < n_blk)
            def _():
                issue_in(j + 2 * _NUM_WORKERS, s0, si0)

            @pl.when(j + _NUM_WORKERS < n_blk)
            def _():
                wait_in(s1, si1)
                transpose_blk(s1, o1)
                issue_out(j + _NUM_WORKERS, o1, so1)

            wait_out(o0, so0)

            @pl.when(j + _NUM_WORKERS < n_blk)
            def _():
                wait_out(o1, so1)

        if has_tail:
            @pl.when(wid == tail_owner)
            def _():
                pltpu.async_copy(tail_hbm, s0, si0).wait()
                transpose_blk(s0, o0)
                pltpu.async_copy(
                    o0, out_hbm.at[pl.ds((vocab - blk) * dim // 128, o_rows)],
                    si0).wait()

    tail_block = lax.slice(table.T, (0, vocab - blk), (dim, vocab))
    return k(table.T, tail_block).reshape(vocab, dim)


def _ffnn_body(maxlen, sums_ref, len_ref, dummy_ref, w1_ref, b1_ref,
               w2_ref, b2_ref, out_ref):
    lf = len_ref[...].astype(jnp.float32)
    sums = sums_ref[...] - (maxlen - lf) * dummy_ref[...]
    vec = sums / jnp.maximum(lf, 1.0)
    h = jnp.dot(vec, w1_ref[...], preferred_element_type=jnp.float32)
    h = jnp.maximum(h + b1_ref[...], 0.0)
    logits = jnp.dot(h, w2_ref[...], preferred_element_type=jnp.float32)
    logits = logits + b2_ref[...]
    m = jnp.max(logits, axis=1, keepdims=True)
    lse = jnp.log(jnp.sum(jnp.exp(logits - m), axis=1, keepdims=True)) + m
    out_ref[...] = logits - lse


def kernel(inp, lengths, table, W1, b1, W2, b2):
    maxlen, batch = inp.shape
    out_dim = W2.shape[1]

    sums, dummy = _pool_sc(inp.astype(jnp.int32), lengths.astype(jnp.int32),
                           _table_rowmajor(table))

    return pl.pallas_call(
        functools.partial(_ffnn_body, float(maxlen)),
        out_shape=jax.ShapeDtypeStruct((batch, out_dim), jnp.float32),
    )(sums, lengths.reshape(batch, 1), dummy,
      W1, b1.reshape(1, -1), W2, b2.reshape(1, -1))
